# baseline (speedup 1.0000x reference)
import functools

import jax
import jax.numpy as jnp
from jax.experimental import pallas as pl
from jax.experimental.pallas import tpu as pltpu

N_TM = 400
N_TN = 5120


def _prep_body(x_ref, w_ref, asrc_ref, adst_ref,
               h_ref, u_ref, p_ref, v_ref, q_ref, hsum_ref):
    i = pl.program_id(0)
    h = jnp.dot(x_ref[...], w_ref[...], preferred_element_type=jnp.float32)
    h_ref[...] = h
    fsrc = jnp.dot(h, asrc_ref[...], preferred_element_type=jnp.float32)
    fdst = jnp.dot(h, adst_ref[...], preferred_element_type=jnp.float32)
    u_ref[...] = jnp.exp(fsrc)
    p_ref[...] = jnp.exp(0.2 * fsrc)
    v_ref[...] = jnp.exp(fdst)
    q_ref[...] = jnp.exp(0.2 * fdst)

    @pl.when(i == 0)
    def _():
        hsum_ref[...] = jnp.zeros_like(hsum_ref)

    hsum_ref[...] += jnp.sum(h, axis=0, keepdims=True)


def _prep(x, W, a_src, a_dst, tm):
    n, din = x.shape
    dout = W.shape[1]
    grid = (n // tm,)
    return pl.pallas_call(
        _prep_body,
        grid=grid,
        in_specs=[
            pl.BlockSpec((tm, din), lambda i: (i, 0)),
            pl.BlockSpec((din, dout), lambda i: (0, 0)),
            pl.BlockSpec((dout, 1), lambda i: (0, 0)),
            pl.BlockSpec((dout, 1), lambda i: (0, 0)),
        ],
        out_specs=[
            pl.BlockSpec((tm, dout), lambda i: (i, 0)),
            pl.BlockSpec((tm, 1), lambda i: (i, 0)),
            pl.BlockSpec((tm, 1), lambda i: (i, 0)),
            pl.BlockSpec((tm, 1), lambda i: (i, 0)),
            pl.BlockSpec((tm, 1), lambda i: (i, 0)),
            pl.BlockSpec((1, dout), lambda i: (0, 0)),
        ],
        out_shape=[
            jax.ShapeDtypeStruct((n, dout), jnp.float32),
            jax.ShapeDtypeStruct((n, 1), jnp.float32),
            jax.ShapeDtypeStruct((n, 1), jnp.float32),
            jax.ShapeDtypeStruct((n, 1), jnp.float32),
            jax.ShapeDtypeStruct((n, 1), jnp.float32),
            jax.ShapeDtypeStruct((1, dout), jnp.float32),
        ],
    )(x, W, a_src.reshape(-1, 1), a_dst.reshape(-1, 1))


def _gat_body(adj_ref, hx_ref, u_ref, p_ref, v_ref, q_ref, hsum_ref, b_ref,
              o_ref, acc_ref, *, nj, n, dout):
    j = pl.program_id(1)

    @pl.when(j == 0)
    def _():
        acc_ref[...] = jnp.zeros_like(acc_ref)

    w = jnp.maximum(u_ref[...] * v_ref[...], p_ref[...] * q_ref[...])
    w = jnp.where(adj_ref[...] > 0.0, w, 0.0)
    acc_ref[...] += jnp.dot(w, hx_ref[...], preferred_element_type=jnp.float32)

    @pl.when(j == nj - 1)
    def _():
        acc = acc_ref[...]
        rs = acc[:, dout:]
        ok = rs > 0.0
        res = jnp.where(ok, acc[:, :dout] / jnp.where(ok, rs, 1.0),
                        hsum_ref[...] * (1.0 / n))
        o_ref[...] = res + b_ref[...]


def _gat_layer(adj, hx, u, p, v, q, hsum, b, tm, tn):
    n = adj.shape[0]
    nc = hx.shape[0]
    dout = hx.shape[1] - 1
    nj = nc // tn
    grid = (n // tm, nj)
    body = functools.partial(_gat_body, nj=nj, n=n, dout=dout)
    return pl.pallas_call(
        body,
        grid=grid,
        in_specs=[
            pl.BlockSpec((tm, tn), lambda i, j: (i, j)),
            pl.BlockSpec((tn, dout + 1), lambda i, j: (j, 0)),
            pl.BlockSpec((tm, 1), lambda i, j: (i, 0)),
            pl.BlockSpec((tm, 1), lambda i, j: (i, 0)),
            pl.BlockSpec((1, tn), lambda i, j: (0, j)),
            pl.BlockSpec((1, tn), lambda i, j: (0, j)),
            pl.BlockSpec((1, dout), lambda i, j: (0, 0)),
            pl.BlockSpec((1, dout), lambda i, j: (0, 0)),
        ],
        out_specs=pl.BlockSpec((tm, dout), lambda i, j: (i, 0)),
        out_shape=jax.ShapeDtypeStruct((n, dout), jnp.float32),
        scratch_shapes=[
            pltpu.VMEM((tm, dout + 1), jnp.float32),
        ],
        compiler_params=pltpu.CompilerParams(
            dimension_semantics=("parallel", "arbitrary"),
        ),
    )(adj, hx, u, p, v, q, hsum, b.reshape(1, -1))


def _pad_cols(h, v, q, n, tn):
    nc = ((n + tn - 1) // tn) * tn
    hx = jnp.concatenate([h, jnp.ones((n, 1), jnp.float32)], axis=1)
    hx = jnp.pad(hx, ((0, nc - n), (0, 0)))
    vr = jnp.pad(v.reshape(1, -1), ((0, 0), (0, nc - n)))
    qr = jnp.pad(q.reshape(1, -1), ((0, 0), (0, nc - n)))
    return hx, vr, qr


def kernel(x, adj, W1, a1_src, a1_dst, b1, W2, a2_src, a2_dst, b2):
    n = adj.shape[0]
    h1, u1, p1, v1, q1, hs1 = _prep(x, W1, a1_src, a1_dst, N_TM)
    hx1, vr1, qr1 = _pad_cols(h1, v1, q1, n, N_TN)
    out1 = _gat_layer(adj, hx1, u1, p1, vr1, qr1, hs1, b1, N_TM, N_TN)
    h2, u2, p2, v2, q2, hs2 = _prep(out1, W2, a2_src, a2_dst, N_TM)
    hx2, vr2, qr2 = _pad_cols(h2, v2, q2, n, N_TN)
    out2 = _gat_layer(adj, hx2, u2, p2, vr2, qr2, hs2, b2, N_TM, N_TN)
    return out2


# speedup vs baseline: 1.2037x; 1.2037x over previous
import functools

import jax
import jax.numpy as jnp
from jax.experimental import pallas as pl
from jax.experimental.pallas import tpu as pltpu

N_TM = 1000
N_TN = 2560


def _prep_body(x_ref, w_ref, asrc_ref, adst_ref,
               h_ref, u_ref, p_ref, v_ref, q_ref, hsum_ref):
    i = pl.program_id(0)
    h = jnp.dot(x_ref[...], w_ref[...], preferred_element_type=jnp.float32)
    h_ref[...] = h
    fsrc = jnp.dot(h, asrc_ref[...], preferred_element_type=jnp.float32)
    fdst = jnp.dot(h, adst_ref[...], preferred_element_type=jnp.float32)
    u_ref[...] = jnp.exp(fsrc)
    p_ref[...] = jnp.exp(0.2 * fsrc)
    v_ref[...] = jnp.exp(fdst)
    q_ref[...] = jnp.exp(0.2 * fdst)

    @pl.when(i == 0)
    def _():
        hsum_ref[...] = jnp.zeros_like(hsum_ref)

    hsum_ref[...] += jnp.sum(h, axis=0, keepdims=True)


def _prep(x, W, a_src, a_dst, tm):
    n, din = x.shape
    dout = W.shape[1]
    grid = (n // tm,)
    return pl.pallas_call(
        _prep_body,
        grid=grid,
        in_specs=[
            pl.BlockSpec((tm, din), lambda i: (i, 0)),
            pl.BlockSpec((din, dout), lambda i: (0, 0)),
            pl.BlockSpec((dout, 1), lambda i: (0, 0)),
            pl.BlockSpec((dout, 1), lambda i: (0, 0)),
        ],
        out_specs=[
            pl.BlockSpec((tm, dout), lambda i: (i, 0)),
            pl.BlockSpec((tm, 1), lambda i: (i, 0)),
            pl.BlockSpec((tm, 1), lambda i: (i, 0)),
            pl.BlockSpec((tm, 1), lambda i: (i, 0)),
            pl.BlockSpec((tm, 1), lambda i: (i, 0)),
            pl.BlockSpec((1, dout), lambda i: (0, 0)),
        ],
        out_shape=[
            jax.ShapeDtypeStruct((n, dout), jnp.float32),
            jax.ShapeDtypeStruct((n, 1), jnp.float32),
            jax.ShapeDtypeStruct((n, 1), jnp.float32),
            jax.ShapeDtypeStruct((n, 1), jnp.float32),
            jax.ShapeDtypeStruct((n, 1), jnp.float32),
            jax.ShapeDtypeStruct((1, dout), jnp.float32),
        ],
    )(x, W, a_src.reshape(-1, 1), a_dst.reshape(-1, 1))


def _gat_body(adj_ref, hx_ref, u_ref, p_ref, v_ref, q_ref, hsum_ref, b_ref,
              o_ref, acc_ref, *, nj, n, dout):
    j = pl.program_id(1)

    @pl.when(j == 0)
    def _():
        acc_ref[...] = jnp.zeros_like(acc_ref)

    adjb = adj_ref[...].astype(jnp.bfloat16)
    w = jnp.maximum(u_ref[...] * v_ref[...], p_ref[...] * q_ref[...])
    w = jnp.where(adjb > 0, w, jnp.bfloat16(0))
    acc_ref[...] += jnp.dot(w, hx_ref[...], preferred_element_type=jnp.float32)

    @pl.when(j == nj - 1)
    def _():
        acc = acc_ref[...]
        rs = acc[:, dout:]
        ok = rs > 0.0
        res = jnp.where(ok, acc[:, :dout] / jnp.where(ok, rs, 1.0),
                        hsum_ref[...] * (1.0 / n))
        o_ref[...] = res + b_ref[...]


def _gat_layer(adj, hx, u, p, v, q, hsum, b, tm, tn):
    n = adj.shape[0]
    nc = hx.shape[0]
    dout = hx.shape[1] - 1
    nj = nc // tn
    grid = (n // tm, nj)
    body = functools.partial(_gat_body, nj=nj, n=n, dout=dout)
    return pl.pallas_call(
        body,
        grid=grid,
        in_specs=[
            pl.BlockSpec((tm, tn), lambda i, j: (i, j)),
            pl.BlockSpec((tn, dout + 1), lambda i, j: (j, 0)),
            pl.BlockSpec((tm, 1), lambda i, j: (i, 0)),
            pl.BlockSpec((tm, 1), lambda i, j: (i, 0)),
            pl.BlockSpec((1, tn), lambda i, j: (0, j)),
            pl.BlockSpec((1, tn), lambda i, j: (0, j)),
            pl.BlockSpec((1, dout), lambda i, j: (0, 0)),
            pl.BlockSpec((1, dout), lambda i, j: (0, 0)),
        ],
        out_specs=pl.BlockSpec((tm, dout), lambda i, j: (i, 0)),
        out_shape=jax.ShapeDtypeStruct((n, dout), jnp.float32),
        scratch_shapes=[
            pltpu.VMEM((tm, dout + 1), jnp.float32),
        ],
        compiler_params=pltpu.CompilerParams(
            dimension_semantics=("parallel", "arbitrary"),
        ),
    )(adj, hx, u, p, v, q, hsum, b.reshape(1, -1))


def _pad_cols(h, v, q, n, tn):
    nc = ((n + tn - 1) // tn) * tn
    hx = jnp.concatenate([h, jnp.ones((n, 1), jnp.float32)], axis=1)
    hx = jnp.pad(hx, ((0, nc - n), (0, 0))).astype(jnp.bfloat16)
    vr = jnp.pad(v.reshape(1, -1), ((0, 0), (0, nc - n))).astype(jnp.bfloat16)
    qr = jnp.pad(q.reshape(1, -1), ((0, 0), (0, nc - n))).astype(jnp.bfloat16)
    return hx, vr, qr


def kernel(x, adj, W1, a1_src, a1_dst, b1, W2, a2_src, a2_dst, b2):
    n = adj.shape[0]
    h1, u1, p1, v1, q1, hs1 = _prep(x, W1, a1_src, a1_dst, N_TM)
    hx1, vr1, qr1 = _pad_cols(h1, v1, q1, n, N_TN)
    out1 = _gat_layer(adj, hx1, u1.astype(jnp.bfloat16), p1.astype(jnp.bfloat16),
                      vr1, qr1, hs1, b1, N_TM, N_TN)
    h2, u2, p2, v2, q2, hs2 = _prep(out1, W2, a2_src, a2_dst, N_TM)
    hx2, vr2, qr2 = _pad_cols(h2, v2, q2, n, N_TN)
    out2 = _gat_layer(adj, hx2, u2.astype(jnp.bfloat16), p2.astype(jnp.bfloat16),
                      vr2, qr2, hs2, b2, N_TM, N_TN)
    return out2


# speedup vs baseline: 1.2401x; 1.0302x over previous
import functools

import jax
import jax.numpy as jnp
from jax.experimental import pallas as pl
from jax.experimental.pallas import tpu as pltpu

N_TM = 1000
N_TN = 2560


def _prep_body(x_ref, w_ref, asrc_ref, adst_ref,
               h_ref, u_ref, p_ref, v_ref, q_ref, hsum_ref):
    i = pl.program_id(0)
    h = jnp.dot(x_ref[...], w_ref[...], preferred_element_type=jnp.float32)
    h_ref[...] = h
    fsrc = jnp.dot(h, asrc_ref[...], preferred_element_type=jnp.float32)
    fdst = jnp.dot(h, adst_ref[...], preferred_element_type=jnp.float32)
    u_ref[...] = jnp.exp(fsrc)
    p_ref[...] = jnp.exp(0.2 * fsrc)
    v_ref[...] = jnp.exp(fdst)
    q_ref[...] = jnp.exp(0.2 * fdst)

    @pl.when(i == 0)
    def _():
        hsum_ref[...] = jnp.zeros_like(hsum_ref)

    hsum_ref[...] += jnp.sum(h, axis=0, keepdims=True)


def _prep(x, W, a_src, a_dst, tm):
    n, din = x.shape
    dout = W.shape[1]
    grid = (n // tm,)
    return pl.pallas_call(
        _prep_body,
        grid=grid,
        in_specs=[
            pl.BlockSpec((tm, din), lambda i: (i, 0)),
            pl.BlockSpec((din, dout), lambda i: (0, 0)),
            pl.BlockSpec((dout, 1), lambda i: (0, 0)),
            pl.BlockSpec((dout, 1), lambda i: (0, 0)),
        ],
        out_specs=[
            pl.BlockSpec((tm, dout), lambda i: (i, 0)),
            pl.BlockSpec((tm, 1), lambda i: (i, 0)),
            pl.BlockSpec((tm, 1), lambda i: (i, 0)),
            pl.BlockSpec((tm, 1), lambda i: (i, 0)),
            pl.BlockSpec((tm, 1), lambda i: (i, 0)),
            pl.BlockSpec((1, dout), lambda i: (0, 0)),
        ],
        out_shape=[
            jax.ShapeDtypeStruct((n, dout), jnp.float32),
            jax.ShapeDtypeStruct((n, 1), jnp.float32),
            jax.ShapeDtypeStruct((n, 1), jnp.float32),
            jax.ShapeDtypeStruct((n, 1), jnp.float32),
            jax.ShapeDtypeStruct((n, 1), jnp.float32),
            jax.ShapeDtypeStruct((1, dout), jnp.float32),
        ],
    )(x, W, a_src.reshape(-1, 1), a_dst.reshape(-1, 1))


def _gat_body(adj_ref, hx_ref, u_ref, p_ref, v_ref, q_ref, hsum_ref, b_ref,
              o_ref, acc_ref, *, nj, n, dout):
    j = pl.program_id(1)

    @pl.when(j == 0)
    def _():
        acc_ref[...] = jnp.zeros_like(acc_ref)

    adjb = adj_ref[...].astype(jnp.bfloat16)
    w = jnp.maximum(u_ref[...] * v_ref[...], p_ref[...] * q_ref[...])
    w = jnp.where(adjb > 0, w, jnp.bfloat16(0))
    acc_ref[...] += jnp.dot(w, hx_ref[...], preferred_element_type=jnp.float32)

    @pl.when(j == nj - 1)
    def _():
        acc = acc_ref[...]
        rs = acc[:, dout:]
        ok = rs > 0.0
        res = jnp.where(ok, acc[:, :dout] / jnp.where(ok, rs, 1.0),
                        hsum_ref[...] * (1.0 / n))
        o_ref[...] = res + b_ref[...]


def _gat_body_pack(adj_ref, hx_ref, u_ref, p_ref, v_ref, q_ref, hsum_ref,
                   b_ref, o_ref, adj8_ref, acc_ref, *, nj, n, dout):
    adj8_ref[...] = adj_ref[...].astype(jnp.int8)
    _gat_body(adj_ref, hx_ref, u_ref, p_ref, v_ref, q_ref, hsum_ref, b_ref,
              o_ref, acc_ref, nj=nj, n=n, dout=dout)


def _gat_layer(adj, hx, u, p, v, q, hsum, b, tm, tn, pack=False):
    n = adj.shape[0]
    nc = hx.shape[0]
    dout = hx.shape[1] - 1
    nj = nc // tn
    grid = (n // tm, nj)
    out_specs = [pl.BlockSpec((tm, dout), lambda i, j: (i, 0))]
    out_shape = [jax.ShapeDtypeStruct((n, dout), jnp.float32)]
    if pack:
        body = functools.partial(_gat_body_pack, nj=nj, n=n, dout=dout)
        out_specs.append(pl.BlockSpec((tm, tn), lambda i, j: (i, j)))
        out_shape.append(jax.ShapeDtypeStruct((n, n), jnp.int8))
    else:
        body = functools.partial(_gat_body, nj=nj, n=n, dout=dout)
    res = pl.pallas_call(
        body,
        grid=grid,
        in_specs=[
            pl.BlockSpec((tm, tn), lambda i, j: (i, j)),
            pl.BlockSpec((tn, dout + 1), lambda i, j: (j, 0)),
            pl.BlockSpec((tm, 1), lambda i, j: (i, 0)),
            pl.BlockSpec((tm, 1), lambda i, j: (i, 0)),
            pl.BlockSpec((1, tn), lambda i, j: (0, j)),
            pl.BlockSpec((1, tn), lambda i, j: (0, j)),
            pl.BlockSpec((1, dout), lambda i, j: (0, 0)),
            pl.BlockSpec((1, dout), lambda i, j: (0, 0)),
        ],
        out_specs=out_specs,
        out_shape=out_shape,
        scratch_shapes=[
            pltpu.VMEM((tm, dout + 1), jnp.float32),
        ],
        compiler_params=pltpu.CompilerParams(
            dimension_semantics=("parallel", "arbitrary"),
        ),
    )(adj, hx, u, p, v, q, hsum, b.reshape(1, -1))
    return res if pack else res[0]


def _pad_cols(h, v, q, n, tn):
    nc = ((n + tn - 1) // tn) * tn
    hx = jnp.concatenate([h, jnp.ones((n, 1), jnp.float32)], axis=1)
    hx = jnp.pad(hx, ((0, nc - n), (0, 0))).astype(jnp.bfloat16)
    vr = jnp.pad(v.reshape(1, -1), ((0, 0), (0, nc - n))).astype(jnp.bfloat16)
    qr = jnp.pad(q.reshape(1, -1), ((0, 0), (0, nc - n))).astype(jnp.bfloat16)
    return hx, vr, qr


def kernel(x, adj, W1, a1_src, a1_dst, b1, W2, a2_src, a2_dst, b2):
    n = adj.shape[0]
    h1, u1, p1, v1, q1, hs1 = _prep(x, W1, a1_src, a1_dst, N_TM)
    hx1, vr1, qr1 = _pad_cols(h1, v1, q1, n, N_TN)
    out1, adj8 = _gat_layer(adj, hx1, u1.astype(jnp.bfloat16),
                            p1.astype(jnp.bfloat16),
                            vr1, qr1, hs1, b1, N_TM, N_TN, pack=True)
    h2, u2, p2, v2, q2, hs2 = _prep(out1, W2, a2_src, a2_dst, N_TM)
    hx2, vr2, qr2 = _pad_cols(h2, v2, q2, n, N_TN)
    out2 = _gat_layer(adj8, hx2, u2.astype(jnp.bfloat16), p2.astype(jnp.bfloat16),
                      vr2, qr2, hs2, b2, N_TM, N_TN)
    return out2


# speedup vs baseline: 1.2464x; 1.0051x over previous
import functools

import jax
import jax.numpy as jnp
from jax.experimental import pallas as pl
from jax.experimental.pallas import tpu as pltpu

N_TM = 1000
N_TN = 2560


def _prep_body(x_ref, w_ref, asrc_ref, adst_ref,
               h_ref, u_ref, p_ref, v_ref, q_ref, hsum_ref):
    i = pl.program_id(0)
    h = jnp.dot(x_ref[...], w_ref[...], preferred_element_type=jnp.float32)
    h_ref[...] = h
    fsrc = jnp.dot(h, asrc_ref[...], preferred_element_type=jnp.float32)
    fdst = jnp.dot(h, adst_ref[...], preferred_element_type=jnp.float32)
    u_ref[...] = jnp.exp(fsrc)
    p_ref[...] = jnp.exp(0.2 * fsrc)
    v_ref[...] = jnp.exp(fdst)
    q_ref[...] = jnp.exp(0.2 * fdst)

    @pl.when(i == 0)
    def _():
        hsum_ref[...] = jnp.zeros_like(hsum_ref)

    hsum_ref[...] += jnp.sum(h, axis=0, keepdims=True)


def _prep(x, W, a_src, a_dst, tm):
    n, din = x.shape
    dout = W.shape[1]
    grid = (n // tm,)
    return pl.pallas_call(
        _prep_body,
        grid=grid,
        in_specs=[
            pl.BlockSpec((tm, din), lambda i: (i, 0)),
            pl.BlockSpec((din, dout), lambda i: (0, 0)),
            pl.BlockSpec((dout, 1), lambda i: (0, 0)),
            pl.BlockSpec((dout, 1), lambda i: (0, 0)),
        ],
        out_specs=[
            pl.BlockSpec((tm, dout), lambda i: (i, 0)),
            pl.BlockSpec((tm, 1), lambda i: (i, 0)),
            pl.BlockSpec((tm, 1), lambda i: (i, 0)),
            pl.BlockSpec((tm, 1), lambda i: (i, 0)),
            pl.BlockSpec((tm, 1), lambda i: (i, 0)),
            pl.BlockSpec((1, dout), lambda i: (0, 0)),
        ],
        out_shape=[
            jax.ShapeDtypeStruct((n, dout), jnp.float32),
            jax.ShapeDtypeStruct((n, 1), jnp.float32),
            jax.ShapeDtypeStruct((n, 1), jnp.float32),
            jax.ShapeDtypeStruct((n, 1), jnp.float32),
            jax.ShapeDtypeStruct((n, 1), jnp.float32),
            jax.ShapeDtypeStruct((1, dout), jnp.float32),
        ],
    )(x, W, a_src.reshape(-1, 1), a_dst.reshape(-1, 1))


def _gat_body(adj_ref, hx_ref, u_ref, p_ref, v_ref, q_ref, hsum_ref, b_ref,
              o_ref, acc_ref, *, nj, n, dout, mul_mask=False):
    j = pl.program_id(1)

    @pl.when(j == 0)
    def _():
        acc_ref[...] = jnp.zeros_like(acc_ref)

    adjb = adj_ref[...].astype(jnp.bfloat16)
    w = jnp.maximum(u_ref[...] * v_ref[...], p_ref[...] * q_ref[...])
    if mul_mask:
        w = adjb * w
    else:
        w = jnp.where(adjb > 0, w, jnp.bfloat16(0))
    acc_ref[...] += jnp.dot(w, hx_ref[...], preferred_element_type=jnp.float32)

    @pl.when(j == nj - 1)
    def _():
        acc = acc_ref[...]
        rs = acc[:, dout:]
        ok = rs > 0.0
        res = jnp.where(ok, acc[:, :dout] / jnp.where(ok, rs, 1.0),
                        hsum_ref[...] * (1.0 / n))
        o_ref[...] = res + b_ref[...]


def _gat_body_pack(adj_ref, hx_ref, u_ref, p_ref, v_ref, q_ref, hsum_ref,
                   b_ref, o_ref, adj8_ref, acc_ref, *, nj, n, dout):
    adj8_ref[...] = adj_ref[...].astype(jnp.int8)
    _gat_body(adj_ref, hx_ref, u_ref, p_ref, v_ref, q_ref, hsum_ref, b_ref,
              o_ref, acc_ref, nj=nj, n=n, dout=dout)


def _gat_layer(adj, hx, u, p, v, q, hsum, b, tm, tn, pack=False):
    n = adj.shape[0]
    nc = hx.shape[0]
    dout = hx.shape[1] - 1
    nj = nc // tn
    grid = (n // tm, nj)
    out_specs = [pl.BlockSpec((tm, dout), lambda i, j: (i, 0))]
    out_shape = [jax.ShapeDtypeStruct((n, dout), jnp.float32)]
    if pack:
        body = functools.partial(_gat_body_pack, nj=nj, n=n, dout=dout)
        out_specs.append(pl.BlockSpec((tm, tn), lambda i, j: (i, j)))
        out_shape.append(jax.ShapeDtypeStruct((n, n), jnp.int8))
    else:
        body = functools.partial(_gat_body, nj=nj, n=n, dout=dout,
                                 mul_mask=adj.dtype == jnp.int8)
    res = pl.pallas_call(
        body,
        grid=grid,
        in_specs=[
            pl.BlockSpec((tm, tn), lambda i, j: (i, j)),
            pl.BlockSpec((tn, dout + 1), lambda i, j: (j, 0)),
            pl.BlockSpec((tm, 1), lambda i, j: (i, 0)),
            pl.BlockSpec((tm, 1), lambda i, j: (i, 0)),
            pl.BlockSpec((1, tn), lambda i, j: (0, j)),
            pl.BlockSpec((1, tn), lambda i, j: (0, j)),
            pl.BlockSpec((1, dout), lambda i, j: (0, 0)),
            pl.BlockSpec((1, dout), lambda i, j: (0, 0)),
        ],
        out_specs=out_specs,
        out_shape=out_shape,
        scratch_shapes=[
            pltpu.VMEM((tm, dout + 1), jnp.float32),
        ],
        compiler_params=pltpu.CompilerParams(
            dimension_semantics=("parallel", "arbitrary"),
        ),
    )(adj, hx, u, p, v, q, hsum, b.reshape(1, -1))
    return res if pack else res[0]


def _pad_cols(h, v, q, n, tn):
    nc = ((n + tn - 1) // tn) * tn
    hx = jnp.concatenate([h, jnp.ones((n, 1), jnp.float32)], axis=1)
    hx = jnp.pad(hx, ((0, nc - n), (0, 0))).astype(jnp.bfloat16)
    vr = jnp.pad(v.reshape(1, -1), ((0, 0), (0, nc - n))).astype(jnp.bfloat16)
    qr = jnp.pad(q.reshape(1, -1), ((0, 0), (0, nc - n))).astype(jnp.bfloat16)
    return hx, vr, qr


def kernel(x, adj, W1, a1_src, a1_dst, b1, W2, a2_src, a2_dst, b2):
    n = adj.shape[0]
    h1, u1, p1, v1, q1, hs1 = _prep(x, W1, a1_src, a1_dst, N_TM)
    hx1, vr1, qr1 = _pad_cols(h1, v1, q1, n, N_TN)
    out1, adj8 = _gat_layer(adj, hx1, u1.astype(jnp.bfloat16),
                            p1.astype(jnp.bfloat16),
                            vr1, qr1, hs1, b1, N_TM, N_TN, pack=True)
    h2, u2, p2, v2, q2, hs2 = _prep(out1, W2, a2_src, a2_dst, N_TM)
    hx2, vr2, qr2 = _pad_cols(h2, v2, q2, n, N_TN)
    out2 = _gat_layer(adj8, hx2, u2.astype(jnp.bfloat16), p2.astype(jnp.bfloat16),
                      vr2, qr2, hs2, b2, N_TM, N_TN)
    return out2


# speedup vs baseline: 1.2866x; 1.0323x over previous
import functools

import jax
import jax.numpy as jnp
from jax.experimental import pallas as pl
from jax.experimental.pallas import tpu as pltpu

N_TM = 1000
N_TN = 2560
N_TN2 = 5120


def _prep_body(x_ref, w_ref, asrc_ref, adst_ref,
               h_ref, u_ref, p_ref, v_ref, q_ref, hsum_ref):
    i = pl.program_id(0)
    h = jnp.dot(x_ref[...], w_ref[...], preferred_element_type=jnp.float32)
    h_ref[...] = h
    fsrc = jnp.dot(h, asrc_ref[...], preferred_element_type=jnp.float32)
    fdst = jnp.dot(h, adst_ref[...], preferred_element_type=jnp.float32)
    u_ref[...] = jnp.exp(fsrc)
    p_ref[...] = jnp.exp(0.2 * fsrc)
    v_ref[...] = jnp.exp(fdst)
    q_ref[...] = jnp.exp(0.2 * fdst)

    @pl.when(i == 0)
    def _():
        hsum_ref[...] = jnp.zeros_like(hsum_ref)

    hsum_ref[...] += jnp.sum(h, axis=0, keepdims=True)


def _prep(x, W, a_src, a_dst, tm):
    n, din = x.shape
    dout = W.shape[1]
    grid = (n // tm,)
    return pl.pallas_call(
        _prep_body,
        grid=grid,
        in_specs=[
            pl.BlockSpec((tm, din), lambda i: (i, 0)),
            pl.BlockSpec((din, dout), lambda i: (0, 0)),
            pl.BlockSpec((dout, 1), lambda i: (0, 0)),
            pl.BlockSpec((dout, 1), lambda i: (0, 0)),
        ],
        out_specs=[
            pl.BlockSpec((tm, dout), lambda i: (i, 0)),
            pl.BlockSpec((tm, 1), lambda i: (i, 0)),
            pl.BlockSpec((tm, 1), lambda i: (i, 0)),
            pl.BlockSpec((tm, 1), lambda i: (i, 0)),
            pl.BlockSpec((tm, 1), lambda i: (i, 0)),
            pl.BlockSpec((1, dout), lambda i: (0, 0)),
        ],
        out_shape=[
            jax.ShapeDtypeStruct((n, dout), jnp.float32),
            jax.ShapeDtypeStruct((n, 1), jnp.float32),
            jax.ShapeDtypeStruct((n, 1), jnp.float32),
            jax.ShapeDtypeStruct((n, 1), jnp.float32),
            jax.ShapeDtypeStruct((n, 1), jnp.float32),
            jax.ShapeDtypeStruct((1, dout), jnp.float32),
        ],
    )(x, W, a_src.reshape(-1, 1), a_dst.reshape(-1, 1))


def _gat_body(adj_ref, hx_ref, u_ref, p_ref, v_ref, q_ref, hsum_ref, b_ref,
              o_ref, acc_ref, *, nj, n, dout, mul_mask=False):
    j = pl.program_id(1)

    @pl.when(j == 0)
    def _():
        acc_ref[...] = jnp.zeros_like(acc_ref)

    adjb = adj_ref[...].astype(jnp.bfloat16)
    w = jnp.maximum(u_ref[...] * v_ref[...], p_ref[...] * q_ref[...])
    if mul_mask:
        w = adjb * w
    else:
        w = jnp.where(adjb > 0, w, jnp.bfloat16(0))
    acc_ref[...] += jnp.dot(w, hx_ref[...], preferred_element_type=jnp.float32)

    @pl.when(j == nj - 1)
    def _():
        acc = acc_ref[...]
        rs = acc[:, dout:]
        ok = rs > 0.0
        res = jnp.where(ok, acc[:, :dout] / jnp.where(ok, rs, 1.0),
                        hsum_ref[...] * (1.0 / n))
        o_ref[...] = res + b_ref[...]


def _gat_body_pack(adj_ref, hx_ref, u_ref, p_ref, v_ref, q_ref, hsum_ref,
                   b_ref, o_ref, adj8_ref, acc_ref, *, nj, n, dout):
    adj8_ref[...] = adj_ref[...].astype(jnp.int8)
    _gat_body(adj_ref, hx_ref, u_ref, p_ref, v_ref, q_ref, hsum_ref, b_ref,
              o_ref, acc_ref, nj=nj, n=n, dout=dout)


def _gat_layer(adj, hx, u, p, v, q, hsum, b, tm, tn, pack=False):
    n = adj.shape[0]
    nc = hx.shape[0]
    dout = hx.shape[1] - 1
    nj = nc // tn
    grid = (n // tm, nj)
    out_specs = [pl.BlockSpec((tm, dout), lambda i, j: (i, 0))]
    out_shape = [jax.ShapeDtypeStruct((n, dout), jnp.float32)]
    if pack:
        body = functools.partial(_gat_body_pack, nj=nj, n=n, dout=dout)
        out_specs.append(pl.BlockSpec((tm, tn), lambda i, j: (i, j)))
        out_shape.append(jax.ShapeDtypeStruct((n, n), jnp.int8))
    else:
        body = functools.partial(_gat_body, nj=nj, n=n, dout=dout,
                                 mul_mask=adj.dtype == jnp.int8)
    res = pl.pallas_call(
        body,
        grid=grid,
        in_specs=[
            pl.BlockSpec((tm, tn), lambda i, j: (i, j)),
            pl.BlockSpec((tn, dout + 1), lambda i, j: (j, 0)),
            pl.BlockSpec((tm, 1), lambda i, j: (i, 0)),
            pl.BlockSpec((tm, 1), lambda i, j: (i, 0)),
            pl.BlockSpec((1, tn), lambda i, j: (0, j)),
            pl.BlockSpec((1, tn), lambda i, j: (0, j)),
            pl.BlockSpec((1, dout), lambda i, j: (0, 0)),
            pl.BlockSpec((1, dout), lambda i, j: (0, 0)),
        ],
        out_specs=out_specs,
        out_shape=out_shape,
        scratch_shapes=[
            pltpu.VMEM((tm, dout + 1), jnp.float32),
        ],
        compiler_params=pltpu.CompilerParams(
            dimension_semantics=("parallel", "arbitrary"),
        ),
    )(adj, hx, u, p, v, q, hsum, b.reshape(1, -1))
    return res if pack else res[0]


def _pad_cols(h, v, q, n, tn):
    nc = ((n + tn - 1) // tn) * tn
    hx = jnp.concatenate([h, jnp.ones((n, 1), jnp.float32)], axis=1)
    hx = jnp.pad(hx, ((0, nc - n), (0, 0))).astype(jnp.bfloat16)
    vr = jnp.pad(v.reshape(1, -1), ((0, 0), (0, nc - n))).astype(jnp.bfloat16)
    qr = jnp.pad(q.reshape(1, -1), ((0, 0), (0, nc - n))).astype(jnp.bfloat16)
    return hx, vr, qr


def kernel(x, adj, W1, a1_src, a1_dst, b1, W2, a2_src, a2_dst, b2):
    n = adj.shape[0]
    h1, u1, p1, v1, q1, hs1 = _prep(x, W1, a1_src, a1_dst, N_TM)
    hx1, vr1, qr1 = _pad_cols(h1, v1, q1, n, N_TN)
    out1, adj8 = _gat_layer(adj, hx1, u1.astype(jnp.bfloat16),
                            p1.astype(jnp.bfloat16),
                            vr1, qr1, hs1, b1, N_TM, N_TN, pack=True)
    h2, u2, p2, v2, q2, hs2 = _prep(out1, W2, a2_src, a2_dst, N_TM)
    hx2, vr2, qr2 = _pad_cols(h2, v2, q2, n, N_TN2)
    out2 = _gat_layer(adj8, hx2, u2.astype(jnp.bfloat16), p2.astype(jnp.bfloat16),
                      vr2, qr2, hs2, b2, N_TM, N_TN2)
    return out2


# speedup vs baseline: 1.2988x; 1.0095x over previous
import functools

import jax
import jax.numpy as jnp
from jax.experimental import pallas as pl
from jax.experimental.pallas import tpu as pltpu

N_TM = 1000
N_TN = 2560
N_TN2 = 10240


def _prep_body(x_ref, w_ref, asrc_ref, adst_ref,
               h_ref, u_ref, p_ref, v_ref, q_ref, hsum_ref):
    i = pl.program_id(0)
    h = jnp.dot(x_ref[...], w_ref[...], preferred_element_type=jnp.float32)
    h_ref[...] = h
    fsrc = jnp.dot(h, asrc_ref[...], preferred_element_type=jnp.float32)
    fdst = jnp.dot(h, adst_ref[...], preferred_element_type=jnp.float32)
    u_ref[...] = jnp.exp(fsrc)
    p_ref[...] = jnp.exp(0.2 * fsrc)
    v_ref[...] = jnp.exp(fdst)
    q_ref[...] = jnp.exp(0.2 * fdst)

    @pl.when(i == 0)
    def _():
        hsum_ref[...] = jnp.zeros_like(hsum_ref)

    hsum_ref[...] += jnp.sum(h, axis=0, keepdims=True)


def _prep(x, W, a_src, a_dst, tm):
    n, din = x.shape
    dout = W.shape[1]
    grid = (n // tm,)
    return pl.pallas_call(
        _prep_body,
        grid=grid,
        in_specs=[
            pl.BlockSpec((tm, din), lambda i: (i, 0)),
            pl.BlockSpec((din, dout), lambda i: (0, 0)),
            pl.BlockSpec((dout, 1), lambda i: (0, 0)),
            pl.BlockSpec((dout, 1), lambda i: (0, 0)),
        ],
        out_specs=[
            pl.BlockSpec((tm, dout), lambda i: (i, 0)),
            pl.BlockSpec((tm, 1), lambda i: (i, 0)),
            pl.BlockSpec((tm, 1), lambda i: (i, 0)),
            pl.BlockSpec((tm, 1), lambda i: (i, 0)),
            pl.BlockSpec((tm, 1), lambda i: (i, 0)),
            pl.BlockSpec((1, dout), lambda i: (0, 0)),
        ],
        out_shape=[
            jax.ShapeDtypeStruct((n, dout), jnp.float32),
            jax.ShapeDtypeStruct((n, 1), jnp.float32),
            jax.ShapeDtypeStruct((n, 1), jnp.float32),
            jax.ShapeDtypeStruct((n, 1), jnp.float32),
            jax.ShapeDtypeStruct((n, 1), jnp.float32),
            jax.ShapeDtypeStruct((1, dout), jnp.float32),
        ],
    )(x, W, a_src.reshape(-1, 1), a_dst.reshape(-1, 1))


def _gat_body(adj_ref, hx_ref, u_ref, p_ref, v_ref, q_ref, hsum_ref, b_ref,
              o_ref, acc_ref, *, nj, n, dout, mul_mask=False):
    j = pl.program_id(1)

    @pl.when(j == 0)
    def _():
        acc_ref[...] = jnp.zeros_like(acc_ref)

    adjb = adj_ref[...].astype(jnp.bfloat16)
    w = jnp.maximum(u_ref[...] * v_ref[...], p_ref[...] * q_ref[...])
    if mul_mask:
        w = adjb * w
    else:
        w = jnp.where(adjb > 0, w, jnp.bfloat16(0))
    acc_ref[...] += jnp.dot(w, hx_ref[...], preferred_element_type=jnp.float32)

    @pl.when(j == nj - 1)
    def _():
        acc = acc_ref[...]
        rs = acc[:, dout:]
        ok = rs > 0.0
        res = jnp.where(ok, acc[:, :dout] / jnp.where(ok, rs, 1.0),
                        hsum_ref[...] * (1.0 / n))
        o_ref[...] = res + b_ref[...]


def _gat_body_pack(adj_ref, hx_ref, u_ref, p_ref, v_ref, q_ref, hsum_ref,
                   b_ref, o_ref, adj8_ref, acc_ref, *, nj, n, dout):
    adj8_ref[...] = adj_ref[...].astype(jnp.int8)
    _gat_body(adj_ref, hx_ref, u_ref, p_ref, v_ref, q_ref, hsum_ref, b_ref,
              o_ref, acc_ref, nj=nj, n=n, dout=dout)


def _gat_layer(adj, hx, u, p, v, q, hsum, b, tm, tn, pack=False):
    n = adj.shape[0]
    nc = hx.shape[0]
    dout = hx.shape[1] - 1
    nj = nc // tn
    grid = (n // tm, nj)
    out_specs = [pl.BlockSpec((tm, dout), lambda i, j: (i, 0))]
    out_shape = [jax.ShapeDtypeStruct((n, dout), jnp.float32)]
    if pack:
        body = functools.partial(_gat_body_pack, nj=nj, n=n, dout=dout)
        out_specs.append(pl.BlockSpec((tm, tn), lambda i, j: (i, j)))
        out_shape.append(jax.ShapeDtypeStruct((n, n), jnp.int8))
    else:
        body = functools.partial(_gat_body, nj=nj, n=n, dout=dout,
                                 mul_mask=adj.dtype == jnp.int8)
    res = pl.pallas_call(
        body,
        grid=grid,
        in_specs=[
            pl.BlockSpec((tm, tn), lambda i, j: (i, j)),
            pl.BlockSpec((tn, dout + 1), lambda i, j: (j, 0)),
            pl.BlockSpec((tm, 1), lambda i, j: (i, 0)),
            pl.BlockSpec((tm, 1), lambda i, j: (i, 0)),
            pl.BlockSpec((1, tn), lambda i, j: (0, j)),
            pl.BlockSpec((1, tn), lambda i, j: (0, j)),
            pl.BlockSpec((1, dout), lambda i, j: (0, 0)),
            pl.BlockSpec((1, dout), lambda i, j: (0, 0)),
        ],
        out_specs=out_specs,
        out_shape=out_shape,
        scratch_shapes=[
            pltpu.VMEM((tm, dout + 1), jnp.float32),
        ],
        compiler_params=pltpu.CompilerParams(
            dimension_semantics=("parallel", "arbitrary"),
        ),
    )(adj, hx, u, p, v, q, hsum, b.reshape(1, -1))
    return res if pack else res[0]


def _pad_cols(h, v, q, n, tn):
    nc = ((n + tn - 1) // tn) * tn
    hx = jnp.concatenate([h, jnp.ones((n, 1), jnp.float32)], axis=1)
    hx = jnp.pad(hx, ((0, nc - n), (0, 0))).astype(jnp.bfloat16)
    vr = jnp.pad(v.reshape(1, -1), ((0, 0), (0, nc - n))).astype(jnp.bfloat16)
    qr = jnp.pad(q.reshape(1, -1), ((0, 0), (0, nc - n))).astype(jnp.bfloat16)
    return hx, vr, qr


def kernel(x, adj, W1, a1_src, a1_dst, b1, W2, a2_src, a2_dst, b2):
    n = adj.shape[0]
    h1, u1, p1, v1, q1, hs1 = _prep(x, W1, a1_src, a1_dst, N_TM)
    hx1, vr1, qr1 = _pad_cols(h1, v1, q1, n, N_TN)
    out1, adj8 = _gat_layer(adj, hx1, u1.astype(jnp.bfloat16),
                            p1.astype(jnp.bfloat16),
                            vr1, qr1, hs1, b1, N_TM, N_TN, pack=True)
    h2, u2, p2, v2, q2, hs2 = _prep(out1, W2, a2_src, a2_dst, N_TM)
    hx2, vr2, qr2 = _pad_cols(h2, v2, q2, n, N_TN2)
    out2 = _gat_layer(adj8, hx2, u2.astype(jnp.bfloat16), p2.astype(jnp.bfloat16),
                      vr2, qr2, hs2, b2, N_TM, N_TN2)
    return out2


# speedup vs baseline: 1.3163x; 1.0134x over previous
import functools

import jax
import jax.numpy as jnp
from jax.experimental import pallas as pl
from jax.experimental.pallas import tpu as pltpu

N_TM = 1000
N_TN = 5120
N_TN2 = 10240


def _prep_body(x_ref, w_ref, asrc_ref, adst_ref,
               h_ref, u_ref, p_ref, v_ref, q_ref, hsum_ref):
    i = pl.program_id(0)
    h = jnp.dot(x_ref[...], w_ref[...], preferred_element_type=jnp.float32)
    h_ref[...] = h
    fsrc = jnp.dot(h, asrc_ref[...], preferred_element_type=jnp.float32)
    fdst = jnp.dot(h, adst_ref[...], preferred_element_type=jnp.float32)
    u_ref[...] = jnp.exp(fsrc)
    p_ref[...] = jnp.exp(0.2 * fsrc)
    v_ref[...] = jnp.exp(fdst)
    q_ref[...] = jnp.exp(0.2 * fdst)

    @pl.when(i == 0)
    def _():
        hsum_ref[...] = jnp.zeros_like(hsum_ref)

    hsum_ref[...] += jnp.sum(h, axis=0, keepdims=True)


def _prep(x, W, a_src, a_dst, tm):
    n, din = x.shape
    dout = W.shape[1]
    grid = (n // tm,)
    return pl.pallas_call(
        _prep_body,
        grid=grid,
        in_specs=[
            pl.BlockSpec((tm, din), lambda i: (i, 0)),
            pl.BlockSpec((din, dout), lambda i: (0, 0)),
            pl.BlockSpec((dout, 1), lambda i: (0, 0)),
            pl.BlockSpec((dout, 1), lambda i: (0, 0)),
        ],
        out_specs=[
            pl.BlockSpec((tm, dout), lambda i: (i, 0)),
            pl.BlockSpec((tm, 1), lambda i: (i, 0)),
            pl.BlockSpec((tm, 1), lambda i: (i, 0)),
            pl.BlockSpec((tm, 1), lambda i: (i, 0)),
            pl.BlockSpec((tm, 1), lambda i: (i, 0)),
            pl.BlockSpec((1, dout), lambda i: (0, 0)),
        ],
        out_shape=[
            jax.ShapeDtypeStruct((n, dout), jnp.float32),
            jax.ShapeDtypeStruct((n, 1), jnp.float32),
            jax.ShapeDtypeStruct((n, 1), jnp.float32),
            jax.ShapeDtypeStruct((n, 1), jnp.float32),
            jax.ShapeDtypeStruct((n, 1), jnp.float32),
            jax.ShapeDtypeStruct((1, dout), jnp.float32),
        ],
    )(x, W, a_src.reshape(-1, 1), a_dst.reshape(-1, 1))


def _gat_body(adj_ref, hx_ref, u_ref, p_ref, v_ref, q_ref, hsum_ref, b_ref,
              o_ref, acc_ref, *, nj, n, dout, mul_mask=False):
    j = pl.program_id(1)

    @pl.when(j == 0)
    def _():
        acc_ref[...] = jnp.zeros_like(acc_ref)

    adjb = adj_ref[...].astype(jnp.bfloat16)
    w = jnp.maximum(u_ref[...] * v_ref[...], p_ref[...] * q_ref[...])
    if mul_mask:
        w = adjb * w
    else:
        w = jnp.where(adjb > 0, w, jnp.bfloat16(0))
    acc_ref[...] += jnp.dot(w, hx_ref[...], preferred_element_type=jnp.float32)

    @pl.when(j == nj - 1)
    def _():
        acc = acc_ref[...]
        rs = acc[:, dout:]
        ok = rs > 0.0
        res = jnp.where(ok, acc[:, :dout] / jnp.where(ok, rs, 1.0),
                        hsum_ref[...] * (1.0 / n))
        o_ref[...] = res + b_ref[...]


def _gat_body_pack(adj_ref, hx_ref, u_ref, p_ref, v_ref, q_ref, hsum_ref,
                   b_ref, o_ref, adj8_ref, acc_ref, *, nj, n, dout):
    adj8_ref[...] = adj_ref[...].astype(jnp.int8)
    _gat_body(adj_ref, hx_ref, u_ref, p_ref, v_ref, q_ref, hsum_ref, b_ref,
              o_ref, acc_ref, nj=nj, n=n, dout=dout)


def _gat_layer(adj, hx, u, p, v, q, hsum, b, tm, tn, pack=False):
    n = adj.shape[0]
    nc = hx.shape[0]
    dout = hx.shape[1] - 1
    nj = nc // tn
    grid = (n // tm, nj)
    out_specs = [pl.BlockSpec((tm, dout), lambda i, j: (i, 0))]
    out_shape = [jax.ShapeDtypeStruct((n, dout), jnp.float32)]
    if pack:
        body = functools.partial(_gat_body_pack, nj=nj, n=n, dout=dout)
        out_specs.append(pl.BlockSpec((tm, tn), lambda i, j: (i, j)))
        out_shape.append(jax.ShapeDtypeStruct((n, n), jnp.int8))
    else:
        body = functools.partial(_gat_body, nj=nj, n=n, dout=dout,
                                 mul_mask=adj.dtype == jnp.int8)
    res = pl.pallas_call(
        body,
        grid=grid,
        in_specs=[
            pl.BlockSpec((tm, tn), lambda i, j: (i, j)),
            pl.BlockSpec((tn, dout + 1), lambda i, j: (j, 0)),
            pl.BlockSpec((tm, 1), lambda i, j: (i, 0)),
            pl.BlockSpec((tm, 1), lambda i, j: (i, 0)),
            pl.BlockSpec((1, tn), lambda i, j: (0, j)),
            pl.BlockSpec((1, tn), lambda i, j: (0, j)),
            pl.BlockSpec((1, dout), lambda i, j: (0, 0)),
            pl.BlockSpec((1, dout), lambda i, j: (0, 0)),
        ],
        out_specs=out_specs,
        out_shape=out_shape,
        scratch_shapes=[
            pltpu.VMEM((tm, dout + 1), jnp.float32),
        ],
        compiler_params=pltpu.CompilerParams(
            dimension_semantics=("parallel", "arbitrary"),
        ),
    )(adj, hx, u, p, v, q, hsum, b.reshape(1, -1))
    return res if pack else res[0]


def _pad_cols(h, v, q, n, tn):
    nc = ((n + tn - 1) // tn) * tn
    hx = jnp.concatenate([h, jnp.ones((n, 1), jnp.float32)], axis=1)
    hx = jnp.pad(hx, ((0, nc - n), (0, 0))).astype(jnp.bfloat16)
    vr = jnp.pad(v.reshape(1, -1), ((0, 0), (0, nc - n))).astype(jnp.bfloat16)
    qr = jnp.pad(q.reshape(1, -1), ((0, 0), (0, nc - n))).astype(jnp.bfloat16)
    return hx, vr, qr


def kernel(x, adj, W1, a1_src, a1_dst, b1, W2, a2_src, a2_dst, b2):
    n = adj.shape[0]
    h1, u1, p1, v1, q1, hs1 = _prep(x, W1, a1_src, a1_dst, N_TM)
    hx1, vr1, qr1 = _pad_cols(h1, v1, q1, n, N_TN)
    out1, adj8 = _gat_layer(adj, hx1, u1.astype(jnp.bfloat16),
                            p1.astype(jnp.bfloat16),
                            vr1, qr1, hs1, b1, N_TM, N_TN, pack=True)
    h2, u2, p2, v2, q2, hs2 = _prep(out1, W2, a2_src, a2_dst, N_TM)
    hx2, vr2, qr2 = _pad_cols(h2, v2, q2, n, N_TN2)
    out2 = _gat_layer(adj8, hx2, u2.astype(jnp.bfloat16), p2.astype(jnp.bfloat16),
                      vr2, qr2, hs2, b2, N_TM, N_TN2)
    return out2


# speedup vs baseline: 1.3844x; 1.0518x over previous
import functools

import jax
import jax.numpy as jnp
from jax.experimental import pallas as pl
from jax.experimental.pallas import tpu as pltpu

N_TM = 1000
N_TN = 5120
N_TN2 = 10240


def _prep_body(x_ref, w_ref, asrc_ref, adst_ref,
               h_ref, u_ref, p_ref, v_ref, q_ref, hsum_ref):
    i = pl.program_id(0)
    h = jnp.dot(x_ref[...], w_ref[...], preferred_element_type=jnp.float32)
    h_ref[...] = h.astype(jnp.bfloat16)
    fsrc = jnp.dot(h, asrc_ref[...], preferred_element_type=jnp.float32)
    fdst = jnp.dot(h, adst_ref[...], preferred_element_type=jnp.float32)
    u_ref[...] = jnp.exp(fsrc).astype(jnp.bfloat16)
    p_ref[...] = jnp.exp(0.2 * fsrc).astype(jnp.bfloat16)
    v_ref[...] = jnp.exp(fdst).astype(jnp.bfloat16)
    q_ref[...] = jnp.exp(0.2 * fdst).astype(jnp.bfloat16)

    @pl.when(i == 0)
    def _():
        hsum_ref[...] = jnp.zeros_like(hsum_ref)

    hsum_ref[...] += jnp.sum(h, axis=0, keepdims=True)


def _prep(x, W, a_src, a_dst, tm):
    n, din = x.shape
    dout = W.shape[1]
    grid = (n // tm,)
    return pl.pallas_call(
        _prep_body,
        grid=grid,
        in_specs=[
            pl.BlockSpec((tm, din), lambda i: (i, 0)),
            pl.BlockSpec((din, dout), lambda i: (0, 0)),
            pl.BlockSpec((dout, 1), lambda i: (0, 0)),
            pl.BlockSpec((dout, 1), lambda i: (0, 0)),
        ],
        out_specs=[
            pl.BlockSpec((tm, dout), lambda i: (i, 0)),
            pl.BlockSpec((tm, 1), lambda i: (i, 0)),
            pl.BlockSpec((tm, 1), lambda i: (i, 0)),
            pl.BlockSpec((tm, 1), lambda i: (i, 0)),
            pl.BlockSpec((tm, 1), lambda i: (i, 0)),
            pl.BlockSpec((1, dout), lambda i: (0, 0)),
        ],
        out_shape=[
            jax.ShapeDtypeStruct((n, dout), jnp.bfloat16),
            jax.ShapeDtypeStruct((n, 1), jnp.bfloat16),
            jax.ShapeDtypeStruct((n, 1), jnp.bfloat16),
            jax.ShapeDtypeStruct((n, 1), jnp.bfloat16),
            jax.ShapeDtypeStruct((n, 1), jnp.bfloat16),
            jax.ShapeDtypeStruct((1, dout), jnp.float32),
        ],
    )(x, W, a_src.reshape(-1, 1), a_dst.reshape(-1, 1))


def _gat_body(adj_ref, hx_ref, u_ref, p_ref, v_ref, q_ref, hsum_ref, b_ref,
              o_ref, acc_ref, *, nj, n, dout, mul_mask=False):
    j = pl.program_id(1)

    @pl.when(j == 0)
    def _():
        acc_ref[...] = jnp.zeros_like(acc_ref)

    adjb = adj_ref[...].astype(jnp.bfloat16)
    w = jnp.maximum(u_ref[...] * v_ref[...], p_ref[...] * q_ref[...])
    if mul_mask:
        w = adjb * w
    else:
        w = jnp.where(adjb > 0, w, jnp.bfloat16(0))
    acc_ref[...] += jnp.dot(w, hx_ref[...], preferred_element_type=jnp.float32)

    @pl.when(j == nj - 1)
    def _():
        acc = acc_ref[...]
        rs = acc[:, dout:]
        ok = rs > 0.0
        res = jnp.where(ok, acc[:, :dout] / jnp.where(ok, rs, 1.0),
                        hsum_ref[...] * (1.0 / n))
        o_ref[...] = res + b_ref[...]


def _gat_body_pack(adj_ref, hx_ref, u_ref, p_ref, v_ref, q_ref, hsum_ref,
                   b_ref, o_ref, adj8_ref, acc_ref, *, nj, n, dout):
    adj8_ref[...] = adj_ref[...].astype(jnp.int8)
    _gat_body(adj_ref, hx_ref, u_ref, p_ref, v_ref, q_ref, hsum_ref, b_ref,
              o_ref, acc_ref, nj=nj, n=n, dout=dout)


def _gat_layer(adj, hx, u, p, v, q, hsum, b, tm, tn, pack=False):
    n = adj.shape[0]
    nc = hx.shape[0]
    dout = hx.shape[1] - 1
    nj = nc // tn
    grid = (n // tm, nj)
    out_specs = [pl.BlockSpec((tm, dout), lambda i, j: (i, 0))]
    out_shape = [jax.ShapeDtypeStruct((n, dout), jnp.float32)]
    if pack:
        body = functools.partial(_gat_body_pack, nj=nj, n=n, dout=dout)
        out_specs.append(pl.BlockSpec((tm, tn), lambda i, j: (i, j)))
        out_shape.append(jax.ShapeDtypeStruct((n, n), jnp.int8))
    else:
        body = functools.partial(_gat_body, nj=nj, n=n, dout=dout,
                                 mul_mask=adj.dtype == jnp.int8)
    res = pl.pallas_call(
        body,
        grid=grid,
        in_specs=[
            pl.BlockSpec((tm, tn), lambda i, j: (i, j)),
            pl.BlockSpec((tn, dout + 1), lambda i, j: (j, 0)),
            pl.BlockSpec((tm, 1), lambda i, j: (i, 0)),
            pl.BlockSpec((tm, 1), lambda i, j: (i, 0)),
            pl.BlockSpec((1, tn), lambda i, j: (0, j)),
            pl.BlockSpec((1, tn), lambda i, j: (0, j)),
            pl.BlockSpec((1, dout), lambda i, j: (0, 0)),
            pl.BlockSpec((1, dout), lambda i, j: (0, 0)),
        ],
        out_specs=out_specs,
        out_shape=out_shape,
        scratch_shapes=[
            pltpu.VMEM((tm, dout + 1), jnp.float32),
        ],
        compiler_params=pltpu.CompilerParams(
            dimension_semantics=("parallel", "arbitrary"),
        ),
    )(adj, hx, u, p, v, q, hsum, b.reshape(1, -1))
    return res if pack else res[0]


def _pad_cols(h, v, q, n, tn):
    nc = ((n + tn - 1) // tn) * tn
    hx = jnp.concatenate([h, jnp.ones((n, 1), jnp.bfloat16)], axis=1)
    hx = jnp.pad(hx, ((0, nc - n), (0, 0)))
    vr = jnp.pad(v.reshape(1, -1), ((0, 0), (0, nc - n)))
    qr = jnp.pad(q.reshape(1, -1), ((0, 0), (0, nc - n)))
    return hx, vr, qr


def kernel(x, adj, W1, a1_src, a1_dst, b1, W2, a2_src, a2_dst, b2):
    n = adj.shape[0]
    h1, u1, p1, v1, q1, hs1 = _prep(x, W1, a1_src, a1_dst, N_TM)
    hx1, vr1, qr1 = _pad_cols(h1, v1, q1, n, N_TN)
    out1, adj8 = _gat_layer(adj, hx1, u1, p1,
                            vr1, qr1, hs1, b1, N_TM, N_TN, pack=True)
    h2, u2, p2, v2, q2, hs2 = _prep(out1, W2, a2_src, a2_dst, N_TM)
    hx2, vr2, qr2 = _pad_cols(h2, v2, q2, n, N_TN2)
    out2 = _gat_layer(adj8, hx2, u2, p2,
                      vr2, qr2, hs2, b2, N_TM, N_TN2)
    return out2
